# Initial kernel scaffold; baseline (speedup 1.0000x reference)
import functools

import jax
import jax.numpy as jnp
from jax import lax
from jax.experimental import pallas as pl
from jax.experimental.pallas import tpu as pltpu
from jax.experimental.pallas import tpu_sc as plsc

N = 10000
E = 320000
A = 100
D = 128
H = 128
B = 64

NC = 2
NS = 16
NW = NC * NS
EPT = E // NW
CH = 80
NCHUNK = EPT // CH
ZR = 125
NZCH = N // ZR

_HI = jax.lax.Precision.HIGHEST


def _edge_body(y_hbm, vx_hbm, src_hbm, dst_hbm, zeros_hbm, agg_hbm,
               sidx, didx, yrows, vrows, zbuf, acc, sem1, sem2):
    c = lax.axis_index("c")
    s = lax.axis_index("s")
    wid = c * NS + s

    pltpu.sync_copy(zeros_hbm, zbuf)

    @pl.loop(s, NZCH, step=NS)
    def _zero(k):
        pltpu.sync_copy(zbuf, acc.at[pl.ds(k * ZR, ZR)])

    plsc.subcore_barrier()

    base_e = wid * EPT

    @pl.loop(0, NCHUNK)
    def _chunk(j):
        off = base_e + j * CH
        pltpu.sync_copy(src_hbm.at[pl.ds(off, CH)], sidx)
        pltpu.sync_copy(dst_hbm.at[pl.ds(off, CH)], didx)
        g1 = pltpu.async_copy(y_hbm.at[sidx], yrows, sem1)
        g2 = pltpu.async_copy(vx_hbm.at[didx], vrows, sem2)
        g1.wait()
        g2.wait()

        @pl.loop(0, CH)
        def _row(r):
            for k8 in range(H // 16):
                sl = pl.ds(k8 * 16, 16)
                yrows[r, sl] = jnp.maximum(yrows[r, sl] + vrows[r, sl], 0.0)

        pltpu.sync_copy(yrows, acc.at[didx], add=True)

    plsc.subcore_barrier()

    @pl.loop(s, NZCH, step=NS)
    def _wb(k):
        pltpu.sync_copy(acc.at[pl.ds(k * ZR, ZR)], zbuf)
        pltpu.sync_copy(zbuf, agg_hbm.at[c, pl.ds(k * ZR, ZR)])


_edge_kernel = functools.partial(
    pl.kernel,
    mesh=plsc.VectorSubcoreMesh(core_axis_name="c", subcore_axis_name="s"),
    out_type=jax.ShapeDtypeStruct((NC, N, H), jnp.float32),
    scratch_types=[
        pltpu.VMEM((CH,), jnp.int32),
        pltpu.VMEM((CH,), jnp.int32),
        pltpu.VMEM((CH, H), jnp.float32),
        pltpu.VMEM((CH, H), jnp.float32),
        pltpu.VMEM((ZR, H), jnp.float32),
        pltpu.VMEM_SHARED((N, H), jnp.float32),
        pltpu.SemaphoreType.DMA,
        pltpu.SemaphoreType.DMA,
    ],
)(_edge_body)


def _prep_body(xidx_ref, emb_ref, vx_ref, y0_ref):
    iota = lax.broadcasted_iota(jnp.int32, (N, A), 1)
    oh = (iota == xidx_ref[...]).astype(jnp.float32)
    vx = jnp.dot(oh, emb_ref[...], precision=_HI,
                 preferred_element_type=jnp.float32)
    vx_ref[...] = vx
    y0_ref[...] = vx * 2.0


def _bn_relu(z, g, bt):
    m = jnp.mean(z, axis=0, keepdims=True)
    zc = z - m
    v = jnp.mean(zc * zc, axis=0, keepdims=True)
    return jnp.maximum(zc * (g / jnp.sqrt(v + 1e-5)) + bt, 0.0)


def _dense_body(x_ref, agg_ref, vx_ref, w1, b1, g1, t1, w2, b2, g2, t2,
                xo_ref, yo_ref):
    h = x_ref[...] + agg_ref[0] + agg_ref[1]
    z = jnp.dot(h, w1[...], precision=_HI,
                preferred_element_type=jnp.float32) + b1[...]
    r = _bn_relu(z, g1[...], t1[...])
    z2 = jnp.dot(r, w2[...], precision=_HI,
                 preferred_element_type=jnp.float32) + b2[...]
    x_out = _bn_relu(z2, g2[...], t2[...])
    xo_ref[...] = x_out
    yo_ref[...] = x_out + vx_ref[...]


def _pool_body(x_ref, batch_ref, w1, b1, w2, b2, out_ref):
    iota = lax.broadcasted_iota(jnp.int32, (B, N), 0)
    oh = (iota == batch_ref[...]).astype(jnp.float32)
    pooled = jnp.dot(oh, x_ref[...], precision=_HI,
                     preferred_element_type=jnp.float32)
    hh = jnp.maximum(
        jnp.dot(pooled, w1[...], precision=_HI,
                preferred_element_type=jnp.float32) + b1[...], 0.0)
    out_ref[...] = jnp.dot(hh, w2[...], precision=_HI,
                           preferred_element_type=jnp.float32) + b2[...]


_prep = pl.pallas_call(
    _prep_body,
    out_shape=(jax.ShapeDtypeStruct((N, D), jnp.float32),
               jax.ShapeDtypeStruct((N, D), jnp.float32)),
)

_dense = pl.pallas_call(
    _dense_body,
    out_shape=(jax.ShapeDtypeStruct((N, H), jnp.float32),
               jax.ShapeDtypeStruct((N, H), jnp.float32)),
)

_pool = pl.pallas_call(
    _pool_body,
    out_shape=jax.ShapeDtypeStruct((B, 10), jnp.float32),
)


def kernel(x_idx, edge_index, batch, emb, convs, lin1_W, lin1_b, lin2_W,
           lin2_b):
    src = edge_index[0]
    dst = edge_index[1]
    zeros = jnp.zeros((ZR, H), jnp.float32)
    batch2d = batch.reshape(1, N)

    vx, y = _prep(x_idx, emb)
    x = vx
    for p in convs:
        agg = _edge_kernel(y, vx, src, dst, zeros)
        x, y = _dense(x, agg, vx, p['W1'], p['b1'], p['g1'], p['bt1'],
                      p['W2'], p['b2'], p['g2'], p['bt2'])
    return _pool(x, batch2d, lin1_W, lin1_b, lin2_W, lin2_b)


# speedup vs baseline: 3.7642x; 3.7642x over previous
import functools

import jax
import jax.numpy as jnp
from jax import lax
from jax.experimental import pallas as pl
from jax.experimental.pallas import tpu as pltpu
from jax.experimental.pallas import tpu_sc as plsc

N = 10000
E = 320000
A = 100
D = 128
H = 128
B = 64

NC = 2
NS = 16
NW = NC * NS
EPT = E // NW
CH = 80
NCHUNK = EPT // CH
ZR = 80
NZCH = N // ZR

_HI = jax.lax.Precision.HIGHEST


def _edge_body(y_hbm, vx_hbm, src_hbm, dst_hbm, zeros_hbm, agg_hbm,
               sidx, didx, yrows, vrows, zbuf, acc, sem1, sem2):
    c = lax.axis_index("c")
    s = lax.axis_index("s")
    wid = c * NS + s

    pltpu.sync_copy(zeros_hbm, zbuf)

    @pl.loop(s, NZCH, step=NS)
    def _zero(k):
        pltpu.sync_copy(zbuf, acc.at[pl.ds(k * ZR, ZR)])

    plsc.subcore_barrier()

    base_e = wid * EPT

    @pl.loop(0, NCHUNK)
    def _chunk(j):
        off = base_e + j * CH
        pltpu.sync_copy(src_hbm.at[pl.ds(off, CH)], sidx)
        pltpu.sync_copy(dst_hbm.at[pl.ds(off, CH)], didx)
        g1 = pltpu.async_copy(y_hbm.at[sidx], yrows, sem1)
        g2 = pltpu.async_copy(vx_hbm.at[didx], vrows, sem2)
        g1.wait()
        g2.wait()

        @pl.loop(0, CH)
        def _row(r):
            for k8 in range(H // 16):
                sl = pl.ds(k8 * 16, 16)
                yrows[r, sl] = jnp.maximum(yrows[r, sl] + vrows[r, sl], 0.0)

        pltpu.sync_copy(yrows, acc.at[didx], add=True)

    plsc.subcore_barrier()

    @pl.loop(s, NZCH, step=NS)
    def _wb(k):
        pltpu.sync_copy(acc.at[pl.ds(k * ZR, ZR)], zbuf)
        pltpu.sync_copy(zbuf, agg_hbm.at[c, pl.ds(k * ZR, ZR)])


_edge_kernel = functools.partial(
    pl.kernel,
    mesh=plsc.VectorSubcoreMesh(core_axis_name="c", subcore_axis_name="s"),
    out_type=jax.ShapeDtypeStruct((NC, N, H), jnp.float32),
    scratch_types=[
        pltpu.VMEM((CH,), jnp.int32),
        pltpu.VMEM((CH,), jnp.int32),
        pltpu.VMEM((CH, H), jnp.float32),
        pltpu.VMEM((CH, H), jnp.float32),
        pltpu.VMEM((ZR, H), jnp.float32),
        pltpu.VMEM_SHARED((N, H), jnp.float32),
        pltpu.SemaphoreType.DMA,
        pltpu.SemaphoreType.DMA,
    ],
)(_edge_body)


def _prep_body(xidx_ref, emb_ref, vx_ref, y0_ref):
    iota = lax.broadcasted_iota(jnp.int32, (N, A), 1)
    oh = (iota == xidx_ref[...]).astype(jnp.float32)
    vx = jnp.dot(oh, emb_ref[...], precision=_HI,
                 preferred_element_type=jnp.float32)
    vx_ref[...] = vx
    y0_ref[...] = vx * 2.0


def _bn_relu(z, g, bt):
    m = jnp.mean(z, axis=0, keepdims=True)
    zc = z - m
    v = jnp.mean(zc * zc, axis=0, keepdims=True)
    return jnp.maximum(zc * (g / jnp.sqrt(v + 1e-5)) + bt, 0.0)


def _dense_body(x_ref, agg_ref, vx_ref, w1, b1, g1, t1, w2, b2, g2, t2,
                xo_ref, yo_ref):
    h = x_ref[...] + agg_ref[0] + agg_ref[1]
    z = jnp.dot(h, w1[...], precision=_HI,
                preferred_element_type=jnp.float32) + b1[...]
    r = _bn_relu(z, g1[...], t1[...])
    z2 = jnp.dot(r, w2[...], precision=_HI,
                 preferred_element_type=jnp.float32) + b2[...]
    x_out = _bn_relu(z2, g2[...], t2[...])
    xo_ref[...] = x_out
    yo_ref[...] = x_out + vx_ref[...]


def _pool_body(x_ref, batch_ref, w1, b1, w2, b2, out_ref):
    iota = lax.broadcasted_iota(jnp.int32, (B, N), 0)
    oh = (iota == batch_ref[...]).astype(jnp.float32)
    pooled = jnp.dot(oh, x_ref[...], precision=_HI,
                     preferred_element_type=jnp.float32)
    hh = jnp.maximum(
        jnp.dot(pooled, w1[...], precision=_HI,
                preferred_element_type=jnp.float32) + b1[...], 0.0)
    out_ref[...] = jnp.dot(hh, w2[...], precision=_HI,
                           preferred_element_type=jnp.float32) + b2[...]


_prep = pl.pallas_call(
    _prep_body,
    out_shape=(jax.ShapeDtypeStruct((N, D), jnp.float32),
               jax.ShapeDtypeStruct((N, D), jnp.float32)),
)

_dense = pl.pallas_call(
    _dense_body,
    out_shape=(jax.ShapeDtypeStruct((N, H), jnp.float32),
               jax.ShapeDtypeStruct((N, H), jnp.float32)),
)

_pool = pl.pallas_call(
    _pool_body,
    out_shape=jax.ShapeDtypeStruct((B, 10), jnp.float32),
)


def kernel(x_idx, edge_index, batch, emb, convs, lin1_W, lin1_b, lin2_W,
           lin2_b):
    src = edge_index[0]
    dst = edge_index[1]
    zeros = jnp.zeros((ZR, H), jnp.float32)
    batch2d = batch.reshape(1, N)

    vx, y = _prep(x_idx, emb)
    x = vx
    for p in convs:
        agg = _edge_kernel(y, vx, src, dst, zeros)
        x, y = _dense(x, agg, vx, p['W1'], p['b1'], p['g1'], p['bt1'],
                      p['W2'], p['b2'], p['g2'], p['bt2'])
    return _pool(x, batch2d, lin1_W, lin1_b, lin2_W, lin2_b)
